# baseline (speedup 1.0000x reference)
import jax
import jax.numpy as jnp
from jax import lax
from jax.experimental import pallas as pl
from jax.experimental.pallas import tpu as pltpu
from jax.experimental.pallas import tpu_sc as plsc

_BATCH = 16384
_IN_F = 512
_OUT_F = 64
_NC = 2
_NS = 16
_NW = _NC * _NS
_ROWS_PER_W = _BATCH // _NW
_R = 64
_CHUNKS = _ROWS_PER_W // _R
_L = 16
_NG = _OUT_F // _L


def _body(x_hbm, cols_hbm, out_hbm, cols_v, x_v, out_v):
    wid = lax.axis_index("s") * _NC + lax.axis_index("c")
    pltpu.sync_copy(cols_hbm, cols_v)
    colv = [cols_v[pl.ds(g * _L, _L)] for g in range(_NG)]
    row0 = wid * _ROWS_PER_W
    for c in range(_CHUNKS):
        base = row0 + c * _R
        pltpu.sync_copy(x_hbm.at[pl.ds(base * _IN_F, _R * _IN_F)], x_v)

        def row_body(r, carry):
            rb = r * _IN_F
            ob = r * _OUT_F
            for g in range(_NG):
                out_v[pl.ds(ob + g * _L, _L)] = plsc.load_gather(
                    x_v, [colv[g] + rb])
            return carry

        lax.fori_loop(0, _R, row_body, 0)
        pltpu.sync_copy(out_v, out_hbm.at[pl.ds(base * _OUT_F, _R * _OUT_F)])


def kernel(x, columns):
    mesh = plsc.VectorSubcoreMesh(core_axis_name="c", subcore_axis_name="s")
    out_flat = pl.kernel(
        _body,
        mesh=mesh,
        compiler_params=pltpu.CompilerParams(needs_layout_passes=False),
        out_type=jax.ShapeDtypeStruct((_BATCH * _OUT_F,), jnp.float32),
        scratch_types=[
            pltpu.VMEM((_OUT_F,), jnp.int32),
            pltpu.VMEM((_R * _IN_F,), jnp.float32),
            pltpu.VMEM((_R * _OUT_F,), jnp.float32),
        ],
    )(x.reshape(-1), columns)
    return out_flat.reshape(_BATCH, _OUT_F)
